# baseline (speedup 1.0000x reference)
import functools

import jax
import jax.numpy as jnp
from jax import lax
from jax.experimental import pallas as pl
from jax.experimental.pallas import tpu as pltpu
from jax.experimental.pallas import tpu_sc as plsc

V = 8192
N = 32
NCHUNK = V // 16

_NC = 2
_NS = 16

_LN2 = 0.6931471805599453


def _vln(x):
  bi = plsc.bitcast(x, jnp.int32)
  e = jnp.right_shift(bi, 23) - 127
  mb = jnp.bitwise_or(jnp.bitwise_and(bi, 0x007FFFFF), 0x3F800000)
  mf = plsc.bitcast(mb, jnp.float32)
  t = (mf - 1.0) / (mf + 1.0)
  t2 = t * t
  ln_m = 2.0 * t * (1.0 + t2 * (1.0 / 3.0 + t2 * (0.2 + t2 * (1.0 / 7.0
                                                              + t2 / 9.0))))
  return e.astype(jnp.float32) * _LN2 + ln_m


def _body(table_hbm, x_hbm, y_hbm, out_hbm, nll_hbm,
          xv, yv, idxbuf, row, nllv, sem, sem2):
  c = lax.axis_index("c")
  s = lax.axis_index("s")
  w = c * _NS + s
  pltpu.sync_copy(x_hbm, xv)
  pltpu.sync_copy(y_hbm, yv)
  lanes = lax.iota(jnp.int32, 16)
  csplat = jnp.full((16,), c, jnp.int32)
  lane_w = lanes == jnp.full((16,), s, jnp.int32)
  half_x = jnp.where(csplat == 0, xv[pl.ds(0, 16)], xv[pl.ds(16, 16)])
  plsc.store_compressed(idxbuf.at[pl.ds(0, 16)], half_x, mask=lane_w)
  pltpu.async_copy(table_hbm.at[idxbuf.at[pl.ds(0, 1)]], row, sem).wait()
  out_cp = pltpu.async_copy(row, out_hbm.at[pl.ds(w, 1)], sem2)

  half_y = jnp.where(csplat == 0, yv[pl.ds(0, 16)], yv[pl.ds(16, 16)])
  yw = jnp.sum(jnp.where(lane_w, half_y, 0))

  def max_step(j, cur):
    return jnp.maximum(cur, row[0, pl.ds(j * 16, 16)])
  mx = lax.fori_loop(0, NCHUNK, max_step, jnp.full((16,), -jnp.inf,
                                                   jnp.float32))
  m = jnp.max(mx)
  msplat = jnp.broadcast_to(m, (16,))

  def sum_step(j, acc):
    return acc + jnp.exp(row[0, pl.ds(j * 16, 16)] - msplat)
  sv = lax.fori_loop(0, NCHUNK, sum_step, jnp.zeros((16,), jnp.float32))
  lsev = msplat + _vln(jnp.broadcast_to(jnp.sum(sv), (16,)))

  tchunk = row[0, pl.ds(jnp.bitwise_and(yw, -16), 16)]
  tgt = jnp.sum(jnp.where(lanes == jnp.broadcast_to(jnp.bitwise_and(yw, 15),
                                                    (16,)),
                          tchunk, 0.0))
  nllv[...] = lsev - jnp.broadcast_to(tgt, (16,))
  pltpu.sync_copy(nllv.at[pl.ds(0, 16)], nll_hbm.at[w])
  out_cp.wait()


@functools.lru_cache(maxsize=1)
def _make_kernel():
  return pl.kernel(
      _body,
      mesh=plsc.VectorSubcoreMesh(
          core_axis_name="c", subcore_axis_name="s",
          num_cores=_NC, num_subcores=_NS),
      out_type=(
          jax.ShapeDtypeStruct((N, V), jnp.float32),
          jax.ShapeDtypeStruct((N, 16), jnp.float32),
      ),
      compiler_params=pltpu.CompilerParams(needs_layout_passes=False),
      scratch_types=[
          pltpu.VMEM((N,), jnp.int32),
          pltpu.VMEM((N,), jnp.int32),
          pltpu.VMEM((16,), jnp.int32),
          pltpu.VMEM((1, V), jnp.float32),
          pltpu.VMEM((16,), jnp.float32),
          pltpu.SemaphoreType.DMA,
          pltpu.SemaphoreType.DMA,
      ],
  )


def kernel(x, y, table):
  xf = x.reshape(N).astype(jnp.int32)
  yf = y.reshape(N).astype(jnp.int32)
  logits, nll = _make_kernel()(table, xf, yf)
  loss = jnp.mean(nll[:, 0])
  return logits, loss
